# baseline (speedup 1.0000x reference)
import functools

import jax
import jax.numpy as jnp
from jax import lax
from jax.experimental import pallas as pl
from jax.experimental.pallas import tpu as pltpu
from jax.experimental.pallas import tpu_sc as plsc

N_IN = 65536
D = 64
OUT_ROWS = 1000000
N_SC = 2
N_SUB = 16
CHUNK_IDX = N_IN // N_SUB
LOGW = 14
WIN = 1 << LOGW
ROWS_PT = WIN // N_SUB
N_WIN = (OUT_ROWS + WIN - 1) // WIN
FULL_PASSES = N_WIN // 2 - 1
LAST_ROWS = OUT_ROWS - (N_WIN - 1) * WIN
LAST_PT = 32
LAST_PT_LAST = LAST_ROWS - 15 * LAST_PT
CHUNK = 128
SEL_CAP = CHUNK_IDX + CHUNK
WIN_ALLOC = WIN + 16
N_BINS = 64


def _body(in_hbm, idx_hbm, zeros_hbm, out_hbm,
          idx_v, zeros_v, bin_idx, bin_pos, hist, cursor, bstart,
          idx2d, pos2d, staging, window, sem):
    c = lax.axis_index("c")
    s = lax.axis_index("s")

    pltpu.sync_copy(idx_hbm.at[pl.ds(s * CHUNK_IDX, CHUNK_IDX)], idx_v)
    pltpu.sync_copy(zeros_hbm, zeros_v)

    for k in range(N_BINS // 16):
        hist[pl.ds(k * 16, 16)] = jnp.zeros((16,), jnp.int32)

    def hist_body(i, carry):
        v = idx_v[pl.ds(i * 16, 16)]
        wv = v >> LOGW
        cnt, last = plsc.scan_count(wv)
        plsc.addupdate_scatter(hist, [wv], cnt, mask=last)
        return carry

    lax.fori_loop(0, CHUNK_IDX // 16, hist_body, jnp.int32(0))

    run = jnp.int32(0)
    for k in range(N_BINS // 16):
        h = hist[pl.ds(k * 16, 16)]
        ex = run + plsc.cumsum(h) - h
        bstart[pl.ds(k * 16, 16)] = ex
        cursor[pl.ds(k * 16, 16)] = ex
        run = run + jnp.sum(h)

    def read_bin_scalar(vec_ref, w):
        acc = jnp.int32(0)
        for k in range(N_BINS // 16):
            hv = vec_ref[pl.ds(k * 16, 16)]
            bid = k * 16 + lax.iota(jnp.int32, 16)
            acc = acc + jnp.sum(jnp.where(bid == w, hv, 0))
        return acc

    def sort_body(i, carry):
        v = idx_v[pl.ds(i * 16, 16)]
        wv = v >> LOGW
        lv = v & (WIN - 1)
        posv = s * CHUNK_IDX + i * 16 + lax.iota(jnp.int32, 16)
        cnt, last = plsc.scan_count(wv)
        cur = plsc.load_gather(cursor, [wv])
        slot = cur + cnt - 1
        plsc.store_scatter(bin_idx, [slot], lv)
        plsc.store_scatter(bin_pos, [slot], posv)
        plsc.addupdate_scatter(cursor, [wv], cnt, mask=last)
        return carry

    lax.fori_loop(0, CHUNK_IDX // 16, sort_body, jnp.int32(0))

    def do_pass(w, rows_pt, rows_pt_last):
        if rows_pt == rows_pt_last:
            pltpu.sync_copy(zeros_v.at[pl.ds(0, rows_pt)],
                            window.at[pl.ds(s * rows_pt, rows_pt)])
        else:
            @pl.when(s < N_SUB - 1)
            def _():
                pltpu.sync_copy(zeros_v.at[pl.ds(0, rows_pt)],
                                window.at[pl.ds(s * rows_pt, rows_pt)])

            @pl.when(s == N_SUB - 1)
            def _():
                pltpu.sync_copy(zeros_v.at[pl.ds(0, rows_pt_last)],
                                window.at[pl.ds((N_SUB - 1) * rows_pt,
                                                rows_pt_last)])
        plsc.subcore_barrier()

        sbase = read_bin_scalar(bstart, w)
        n = read_bin_scalar(hist, w)
        n_chunks = (n + CHUNK - 1) >> 7

        def chunk_body(cc, carry):
            for k in range(CHUNK // 16):
                gpos = cc * CHUNK + k * 16
                iv = bin_idx[pl.ds(sbase + gpos, 16)]
                pv = bin_pos[pl.ds(sbase + gpos, 16)]
                lanes = lax.iota(jnp.int32, 16)
                valid = (gpos + lanes) < n
                iv = jnp.where(valid, iv, WIN + lanes)
                pv = jnp.where(valid, pv, 0)
                idx2d[0, pl.ds(k * 16, 16)] = iv
                pos2d[0, pl.ds(k * 16, 16)] = pv
            pltpu.async_copy(in_hbm.at[pos2d.at[0]], staging, sem).wait()
            pltpu.sync_copy(staging, window.at[idx2d.at[0]], add=True)
            return carry

        lax.fori_loop(0, n_chunks, chunk_body, jnp.int32(0))
        plsc.subcore_barrier()

        lo = w * WIN
        if rows_pt == rows_pt_last:
            pltpu.sync_copy(window.at[pl.ds(s * rows_pt, rows_pt)],
                            out_hbm.at[pl.ds(lo + s * rows_pt, rows_pt)])
        else:
            @pl.when(s < N_SUB - 1)
            def _():
                pltpu.sync_copy(window.at[pl.ds(s * rows_pt, rows_pt)],
                                out_hbm.at[pl.ds(lo + s * rows_pt, rows_pt)])

            @pl.when(s == N_SUB - 1)
            def _():
                pltpu.sync_copy(
                    window.at[pl.ds((N_SUB - 1) * rows_pt, rows_pt_last)],
                    out_hbm.at[pl.ds(lo + (N_SUB - 1) * rows_pt,
                                     rows_pt_last)])

    def pass_body(p, carry):
        do_pass(p * N_SC + c, ROWS_PT, ROWS_PT)
        return carry

    lax.fori_loop(0, FULL_PASSES, pass_body, jnp.int32(0))

    @pl.when(c == 0)
    def _():
        do_pass(jnp.int32(N_WIN - 2), ROWS_PT, ROWS_PT)

    @pl.when(c == 1)
    def _():
        do_pass(jnp.int32(N_WIN - 1), LAST_PT, LAST_PT_LAST)


@jax.jit
def _scatter(input_, idx32, zeros):
    mesh = plsc.VectorSubcoreMesh(core_axis_name="c", subcore_axis_name="s")
    k = functools.partial(
        pl.kernel,
        out_type=jax.ShapeDtypeStruct((OUT_ROWS, D), jnp.bfloat16),
        mesh=mesh,
        compiler_params=pltpu.CompilerParams(needs_layout_passes=False, use_tc_tiling_on_sc=False),
        scratch_types=[
            pltpu.VMEM((CHUNK_IDX,), jnp.int32),
            pltpu.VMEM((ROWS_PT, D), jnp.bfloat16),
            pltpu.VMEM((SEL_CAP,), jnp.int32),
            pltpu.VMEM((SEL_CAP,), jnp.int32),
            pltpu.VMEM((N_BINS,), jnp.int32),
            pltpu.VMEM((N_BINS,), jnp.int32),
            pltpu.VMEM((N_BINS,), jnp.int32),
            pltpu.VMEM((1, CHUNK), jnp.int32),
            pltpu.VMEM((1, CHUNK), jnp.int32),
            pltpu.VMEM((CHUNK, D), jnp.bfloat16),
            pltpu.VMEM_SHARED((WIN_ALLOC, D), jnp.bfloat16),
            pltpu.SemaphoreType.DMA,
        ],
    )(_body)
    return k(input_, idx32, zeros)


def kernel(input_, indices, output_size, n_tpc):
    idx32 = indices.astype(jnp.int32)
    zeros = jnp.zeros((ROWS_PT, D), jnp.bfloat16)
    return _scatter(input_, idx32, zeros)


# speedup vs baseline: 1.0072x; 1.0072x over previous
import functools

import jax
import jax.numpy as jnp
from jax import lax
from jax.experimental import pallas as pl
from jax.experimental.pallas import tpu as pltpu
from jax.experimental.pallas import tpu_sc as plsc

N_IN = 65536
D = 64
OUT_ROWS = 1000000
N_SC = 2
N_SUB = 16
CHUNK_IDX = N_IN // N_SUB
LOGW = 14
WIN = 1 << LOGW
ROWS_PT = WIN // N_SUB
N_WIN = (OUT_ROWS + WIN - 1) // WIN
FULL_PASSES = N_WIN // 2 - 1
LAST_ROWS = OUT_ROWS - (N_WIN - 1) * WIN
LAST_PT = 32
LAST_PT_LAST = LAST_ROWS - 15 * LAST_PT
CHUNK = 128
SEL_CAP = CHUNK_IDX + CHUNK
WIN_ALLOC = WIN + 16
N_BINS = 64


def _body(in_hbm, idx_hbm, zeros_hbm, out_hbm,
          idx_v, zeros_v, bin_idx, bin_pos, hist, cursor, bstart,
          idx2d, pos2d, staging, window, sem, sem_d):
    c = lax.axis_index("c")
    s = lax.axis_index("s")

    pltpu.sync_copy(idx_hbm.at[pl.ds(s * CHUNK_IDX, CHUNK_IDX)], idx_v)
    pltpu.sync_copy(zeros_hbm, zeros_v)

    for k in range(N_BINS // 16):
        hist[pl.ds(k * 16, 16)] = jnp.zeros((16,), jnp.int32)

    def hist_body(i, carry):
        v = idx_v[pl.ds(i * 16, 16)]
        wv = v >> LOGW
        cnt, last = plsc.scan_count(wv)
        plsc.addupdate_scatter(hist, [wv], cnt, mask=last)
        return carry

    lax.fori_loop(0, CHUNK_IDX // 16, hist_body, jnp.int32(0))

    run = jnp.int32(0)
    for k in range(N_BINS // 16):
        h = hist[pl.ds(k * 16, 16)]
        ex = run + plsc.cumsum(h) - h
        bstart[pl.ds(k * 16, 16)] = ex
        cursor[pl.ds(k * 16, 16)] = ex
        run = run + jnp.sum(h)

    def read_bin_scalar(vec_ref, w):
        acc = jnp.int32(0)
        for k in range(N_BINS // 16):
            hv = vec_ref[pl.ds(k * 16, 16)]
            bid = k * 16 + lax.iota(jnp.int32, 16)
            acc = acc + jnp.sum(jnp.where(bid == w, hv, 0))
        return acc

    def sort_body(i, carry):
        v = idx_v[pl.ds(i * 16, 16)]
        wv = v >> LOGW
        lv = v & (WIN - 1)
        posv = s * CHUNK_IDX + i * 16 + lax.iota(jnp.int32, 16)
        cnt, last = plsc.scan_count(wv)
        cur = plsc.load_gather(cursor, [wv])
        slot = cur + cnt - 1
        plsc.store_scatter(bin_idx, [slot], lv)
        plsc.store_scatter(bin_pos, [slot], posv)
        plsc.addupdate_scatter(cursor, [wv], cnt, mask=last)
        return carry

    lax.fori_loop(0, CHUNK_IDX // 16, sort_body, jnp.int32(0))

    def build_chunk(cc, sbase, n):
        for k in range(CHUNK // 16):
            gpos = cc * CHUNK + k * 16
            iv = bin_idx[pl.ds(sbase + gpos, 16)]
            pv = bin_pos[pl.ds(sbase + gpos, 16)]
            lanes = lax.iota(jnp.int32, 16)
            valid = (gpos + lanes) < n
            iv = jnp.where(valid, iv, WIN + lanes)
            pv = jnp.where(valid, pv, 0)
            idx2d[0, pl.ds(k * 16, 16)] = iv
            pos2d[0, pl.ds(k * 16, 16)] = pv

    def do_pass(w, rows_pt, rows_pt_last, wait_prev, sync_drain):
        sbase = read_bin_scalar(bstart, w)
        n = read_bin_scalar(hist, w)
        n_chunks = (n + CHUNK - 1) >> 7

        @pl.when(n_chunks > 0)
        def _():
            build_chunk(jnp.int32(0), sbase, n)
            pltpu.async_copy(in_hbm.at[pos2d.at[0]], staging, sem)

        if wait_prev:
            pltpu.make_async_copy(
                window.at[pl.ds(s * ROWS_PT, ROWS_PT)],
                out_hbm.at[pl.ds(s * ROWS_PT, ROWS_PT)],
                sem_d).wait()

        if rows_pt == rows_pt_last:
            pltpu.sync_copy(zeros_v.at[pl.ds(0, rows_pt)],
                            window.at[pl.ds(s * rows_pt, rows_pt)])
        else:
            @pl.when(s < N_SUB - 1)
            def _():
                pltpu.sync_copy(zeros_v.at[pl.ds(0, rows_pt)],
                                window.at[pl.ds(s * rows_pt, rows_pt)])

            @pl.when(s == N_SUB - 1)
            def _():
                pltpu.sync_copy(zeros_v.at[pl.ds(0, rows_pt_last)],
                                window.at[pl.ds((N_SUB - 1) * rows_pt,
                                                rows_pt_last)])
        plsc.subcore_barrier()

        def chunk_body(cc, carry):
            @pl.when(cc > 0)
            def _():
                build_chunk(cc, sbase, n)
                pltpu.async_copy(in_hbm.at[pos2d.at[0]], staging, sem)

            pltpu.make_async_copy(in_hbm.at[pos2d.at[0]], staging, sem).wait()
            pltpu.sync_copy(staging, window.at[idx2d.at[0]], add=True)
            return carry

        lax.fori_loop(0, n_chunks, chunk_body, jnp.int32(0))
        plsc.subcore_barrier()

        lo = w * WIN

        def drain(src_ref, dst_ref):
            if sync_drain:
                pltpu.sync_copy(src_ref, dst_ref)
            else:
                pltpu.async_copy(src_ref, dst_ref, sem_d)

        if rows_pt == rows_pt_last:
            drain(window.at[pl.ds(s * rows_pt, rows_pt)],
                  out_hbm.at[pl.ds(lo + s * rows_pt, rows_pt)])
        else:
            @pl.when(s < N_SUB - 1)
            def _():
                drain(window.at[pl.ds(s * rows_pt, rows_pt)],
                      out_hbm.at[pl.ds(lo + s * rows_pt, rows_pt)])

            @pl.when(s == N_SUB - 1)
            def _():
                drain(window.at[pl.ds((N_SUB - 1) * rows_pt, rows_pt_last)],
                      out_hbm.at[pl.ds(lo + (N_SUB - 1) * rows_pt,
                                       rows_pt_last)])

    def pass_body(p, carry):
        @pl.when(p == 0)
        def _():
            do_pass(p * N_SC + c, ROWS_PT, ROWS_PT, False, False)

        @pl.when(p > 0)
        def _():
            do_pass(p * N_SC + c, ROWS_PT, ROWS_PT, True, False)

        return carry

    lax.fori_loop(0, FULL_PASSES, pass_body, jnp.int32(0))

    @pl.when(c == 0)
    def _():
        do_pass(jnp.int32(N_WIN - 2), ROWS_PT, ROWS_PT, True, True)

    @pl.when(c == 1)
    def _():
        do_pass(jnp.int32(N_WIN - 1), LAST_PT, LAST_PT_LAST, True, True)


@jax.jit
def _scatter(input_, idx32, zeros):
    mesh = plsc.VectorSubcoreMesh(core_axis_name="c", subcore_axis_name="s")
    k = functools.partial(
        pl.kernel,
        out_type=jax.ShapeDtypeStruct((OUT_ROWS, D), jnp.bfloat16),
        mesh=mesh,
        compiler_params=pltpu.CompilerParams(needs_layout_passes=False, use_tc_tiling_on_sc=False),
        scratch_types=[
            pltpu.VMEM((CHUNK_IDX,), jnp.int32),
            pltpu.VMEM((ROWS_PT, D), jnp.bfloat16),
            pltpu.VMEM((SEL_CAP,), jnp.int32),
            pltpu.VMEM((SEL_CAP,), jnp.int32),
            pltpu.VMEM((N_BINS,), jnp.int32),
            pltpu.VMEM((N_BINS,), jnp.int32),
            pltpu.VMEM((N_BINS,), jnp.int32),
            pltpu.VMEM((1, CHUNK), jnp.int32),
            pltpu.VMEM((1, CHUNK), jnp.int32),
            pltpu.VMEM((CHUNK, D), jnp.bfloat16),
            pltpu.VMEM_SHARED((WIN_ALLOC, D), jnp.bfloat16),
            pltpu.SemaphoreType.DMA,
            pltpu.SemaphoreType.DMA,
        ],
    )(_body)
    return k(input_, idx32, zeros)


def kernel(input_, indices, output_size, n_tpc):
    idx32 = indices.astype(jnp.int32)
    zeros = jnp.zeros((ROWS_PT, D), jnp.bfloat16)
    return _scatter(input_, idx32, zeros)
